# Initial kernel scaffold; baseline (speedup 1.0000x reference)
import jax
import jax.numpy as jnp
from jax.experimental import pallas as pl


def kernel(pred, gt, mask):
    raise NotImplementedError("write your pallas kernel here")


# speedup vs baseline: 42.5604x; 42.5604x over previous
import functools

import jax
import jax.numpy as jnp
from jax import lax
from jax.experimental import pallas as pl
from jax.experimental.pallas import tpu as pltpu
from jax.experimental.pallas import tpu_sc as plsc

NC = 2
NS = 16
L = 16
NW = NC * NS

N = 16 * 512 * 512
PER_W = N // NW
CHUNK = 16384
NCHUNK = PER_W // CHUNK

_MESH = plsc.VectorSubcoreMesh(core_axis_name="c", subcore_axis_name="s")


@functools.partial(
    pl.kernel,
    mesh=_MESH,
    out_type=jax.ShapeDtypeStruct((NW, 3 * L), jnp.float32),
    scratch_types=(
        [pltpu.VMEM((CHUNK,), jnp.float32) for _ in range(6)]
        + [pltpu.VMEM((3 * L,), jnp.float32)]
        + [pltpu.SemaphoreType.DMA, pltpu.SemaphoreType.DMA]
    ),
)
def _reduce_pass(pred_h, gt_h, mask_h, out_h,
                 p0, g0, m0, p1, g1, m1, stage, sem0, sem1):
    wid = lax.axis_index("c") * NS + lax.axis_index("s")
    base = wid * PER_W
    bufs = ((p0, g0, m0, sem0), (p1, g1, m1, sem1))

    def start(g):
        pb, gb, mb, sem = bufs[g % 2]
        off = base + g * CHUNK
        return (
            pltpu.async_copy(pred_h.at[pl.ds(off, CHUNK)], pb, sem),
            pltpu.async_copy(gt_h.at[pl.ds(off, CHUNK)], gb, sem),
            pltpu.async_copy(mask_h.at[pl.ds(off, CHUNK)], mb, sem),
        )

    pending = [None, None]
    pending[0] = start(0)
    zero = jnp.zeros((L,), jnp.float32)
    pos_sum, pos_cnt, loss_sum = zero, zero, zero
    for g in range(NCHUNK):
        if g + 1 < NCHUNK:
            pending[(g + 1) % 2] = start(g + 1)
        for h in pending[g % 2]:
            h.wait()
        pb, gb, mb, _ = bufs[g % 2]

        def step(i, carry, pb=pb, gb=gb, mb=mb):
            ps, pc, ls = carry
            sl = pl.ds(i * L, L)
            l = jnp.abs(pb[sl] - gb[sl])
            mk = mb[sl]
            return (ps + l * mk, pc + mk, ls + l)

        pos_sum, pos_cnt, loss_sum = lax.fori_loop(
            0, CHUNK // L, step, (pos_sum, pos_cnt, loss_sum))

    stage[pl.ds(0, L)] = pos_sum
    stage[pl.ds(L, L)] = pos_cnt
    stage[pl.ds(2 * L, L)] = loss_sum
    pltpu.sync_copy(stage, out_h.at[wid])


@functools.partial(
    pl.kernel,
    mesh=_MESH,
    out_type=jax.ShapeDtypeStruct((NW, 3 * L), jnp.float32),
    scratch_types=(
        [pltpu.VMEM((CHUNK,), jnp.float32) for _ in range(6)]
        + [pltpu.VMEM((3 * L,), jnp.float32), pltpu.VMEM((L,), jnp.float32)]
        + [pltpu.SemaphoreType.DMA, pltpu.SemaphoreType.DMA]
    ),
)
def _count_pass(pred_h, gt_h, mask_h, thr_h, out_h,
                p0, g0, m0, p1, g1, m1, stage, thr_v, sem0, sem1):
    wid = lax.axis_index("c") * NS + lax.axis_index("s")
    base = wid * PER_W
    pltpu.sync_copy(thr_h, thr_v)
    thr = thr_v[pl.ds(0, L)]
    bufs = ((p0, g0, m0, sem0), (p1, g1, m1, sem1))

    def start(g):
        pb, gb, mb, sem = bufs[g % 2]
        off = base + g * CHUNK
        return (
            pltpu.async_copy(pred_h.at[pl.ds(off, CHUNK)], pb, sem),
            pltpu.async_copy(gt_h.at[pl.ds(off, CHUNK)], gb, sem),
            pltpu.async_copy(mask_h.at[pl.ds(off, CHUNK)], mb, sem),
        )

    pending = [None, None]
    pending[0] = start(0)
    zero = jnp.zeros((L,), jnp.float32)
    one = jnp.ones((L,), jnp.float32)
    cnt_ge, cnt_gt, sum_gt = zero, zero, zero
    for g in range(NCHUNK):
        if g + 1 < NCHUNK:
            pending[(g + 1) % 2] = start(g + 1)
        for h in pending[g % 2]:
            h.wait()
        pb, gb, mb, _ = bufs[g % 2]

        def step(i, carry, pb=pb, gb=gb, mb=mb):
            cge, cgt, sgt = carry
            sl = pl.ds(i * L, L)
            l = jnp.abs(pb[sl] - gb[sl])
            v = l - l * mb[sl]
            cge = cge + jnp.where(v >= thr, one, zero)
            is_gt = v > thr
            cgt = cgt + jnp.where(is_gt, one, zero)
            sgt = sgt + jnp.where(is_gt, v, zero)
            return (cge, cgt, sgt)

        cnt_ge, cnt_gt, sum_gt = lax.fori_loop(
            0, CHUNK // L, step, (cnt_ge, cnt_gt, sum_gt))

    stage[pl.ds(0, L)] = cnt_ge
    stage[pl.ds(L, L)] = cnt_gt
    stage[pl.ds(2 * L, L)] = sum_gt
    pltpu.sync_copy(stage, out_h.at[wid])


def _probe(pred_f, gt_f, mask_f, bits):
    thr = lax.bitcast_convert_type(bits, jnp.float32)
    out = _count_pass(pred_f, gt_f, mask_f, jnp.full((L,), thr))
    s = out.reshape(NW, 3, L).sum(axis=(0, 2))
    return s[0], s[1], s[2]


def _topk_sum(pred_f, gt_f, mask_f, kf):
    def cond(st):
        lo, hi = st
        return hi - lo > 1

    def body(st):
        lo, hi = st
        mid = lo + (hi - lo) // 2
        cge, _, _ = _probe(pred_f, gt_f, mask_f, mid)
        big = cge >= kf
        return (jnp.where(big, mid, lo), jnp.where(big, hi, mid))

    lo0 = jnp.int32(0)
    hi0 = jnp.int32(0x7F800000)
    lo, _ = lax.while_loop(cond, body, (lo0, hi0))
    _, cgt, sgt = _probe(pred_f, gt_f, mask_f, lo)
    t = lax.bitcast_convert_type(lo, jnp.float32)
    return sgt + (kf - cgt) * t


def kernel(pred, gt, mask):
    pred_f = pred.reshape(-1)
    gt_f = gt.reshape(-1)
    mask_f = mask.reshape(-1)

    part = _reduce_pass(pred_f, gt_f, mask_f).reshape(NW, 3, L)
    sums = part.sum(axis=(0, 2))
    pos_sum, pos_cnt, loss_sum = sums[0], sums[1], sums[2]
    neg_sum = loss_sum - pos_sum
    neg_cnt = jnp.float32(N) - pos_cnt
    kf = jnp.minimum(neg_cnt, jnp.floor(pos_cnt * 3.0))

    neg_topk = lax.cond(
        kf >= neg_cnt,
        lambda: neg_sum,
        lambda: _topk_sum(pred_f, gt_f, mask_f, kf),
    )

    positive_loss = pos_sum / pos_cnt
    negative_loss = neg_topk / kf
    return (positive_loss + negative_loss, positive_loss, negative_loss)


# speedup vs baseline: 46.7599x; 1.0987x over previous
import functools

import jax
import jax.numpy as jnp
from jax import lax
from jax.experimental import pallas as pl
from jax.experimental.pallas import tpu as pltpu
from jax.experimental.pallas import tpu_sc as plsc

NC = 2
NS = 16
L = 16
NW = NC * NS

N = 16 * 512 * 512
PER_W = N // NW
CHUNK = 16384
NCHUNK = PER_W // CHUNK

U = 8

_MESH = plsc.VectorSubcoreMesh(core_axis_name="c", subcore_axis_name="s")


def _tree_add(vs):
    while len(vs) > 1:
        vs = [a + b for a, b in zip(vs[::2], vs[1::2])] + (
            [vs[-1]] if len(vs) % 2 else [])
    return vs[0]


@functools.partial(
    pl.kernel,
    mesh=_MESH,
    out_type=jax.ShapeDtypeStruct((NW, 3 * L), jnp.float32),
    scratch_types=(
        [pltpu.VMEM((CHUNK,), jnp.float32) for _ in range(6)]
        + [pltpu.VMEM((3 * L,), jnp.float32)]
        + [pltpu.SemaphoreType.DMA, pltpu.SemaphoreType.DMA]
    ),
)
def _reduce_pass(pred_h, gt_h, mask_h, out_h,
                 p0, g0, m0, p1, g1, m1, stage, sem0, sem1):
    wid = lax.axis_index("c") * NS + lax.axis_index("s")
    base = wid * PER_W
    bufs = ((p0, g0, m0, sem0), (p1, g1, m1, sem1))

    def start(g):
        pb, gb, mb, sem = bufs[g % 2]
        off = base + g * CHUNK
        return (
            pltpu.async_copy(pred_h.at[pl.ds(off, CHUNK)], pb, sem),
            pltpu.async_copy(gt_h.at[pl.ds(off, CHUNK)], gb, sem),
            pltpu.async_copy(mask_h.at[pl.ds(off, CHUNK)], mb, sem),
        )

    pending = [None, None]
    pending[0] = start(0)
    zero = jnp.zeros((L,), jnp.float32)
    pos_sum, pos_cnt, loss_sum = zero, zero, zero
    for g in range(NCHUNK):
        if g + 1 < NCHUNK:
            pending[(g + 1) % 2] = start(g + 1)
        for h in pending[g % 2]:
            h.wait()
        pb, gb, mb, _ = bufs[g % 2]

        def step(i, carry, pb=pb, gb=gb, mb=mb):
            ps, pc, ls = carry
            base_i = i * (U * L)
            lv, mv, pv = [], [], []
            for j in range(U):
                sl = pl.ds(base_i + j * L, L)
                l = jnp.abs(pb[sl] - gb[sl])
                mk = mb[sl]
                lv.append(l)
                mv.append(mk)
                pv.append(l * mk)
            return (ps + _tree_add(pv), pc + _tree_add(mv), ls + _tree_add(lv))

        pos_sum, pos_cnt, loss_sum = lax.fori_loop(
            0, CHUNK // (U * L), step, (pos_sum, pos_cnt, loss_sum))

    stage[pl.ds(0, L)] = pos_sum
    stage[pl.ds(L, L)] = pos_cnt
    stage[pl.ds(2 * L, L)] = loss_sum
    pltpu.sync_copy(stage, out_h.at[wid])


@functools.partial(
    pl.kernel,
    mesh=_MESH,
    out_type=jax.ShapeDtypeStruct((NW, 3 * L), jnp.float32),
    scratch_types=(
        [pltpu.VMEM((CHUNK,), jnp.float32) for _ in range(6)]
        + [pltpu.VMEM((3 * L,), jnp.float32), pltpu.VMEM((L,), jnp.float32)]
        + [pltpu.SemaphoreType.DMA, pltpu.SemaphoreType.DMA]
    ),
)
def _count_pass(pred_h, gt_h, mask_h, thr_h, out_h,
                p0, g0, m0, p1, g1, m1, stage, thr_v, sem0, sem1):
    wid = lax.axis_index("c") * NS + lax.axis_index("s")
    base = wid * PER_W
    pltpu.sync_copy(thr_h, thr_v)
    thr = thr_v[pl.ds(0, L)]
    bufs = ((p0, g0, m0, sem0), (p1, g1, m1, sem1))

    def start(g):
        pb, gb, mb, sem = bufs[g % 2]
        off = base + g * CHUNK
        return (
            pltpu.async_copy(pred_h.at[pl.ds(off, CHUNK)], pb, sem),
            pltpu.async_copy(gt_h.at[pl.ds(off, CHUNK)], gb, sem),
            pltpu.async_copy(mask_h.at[pl.ds(off, CHUNK)], mb, sem),
        )

    pending = [None, None]
    pending[0] = start(0)
    zero = jnp.zeros((L,), jnp.float32)
    one = jnp.ones((L,), jnp.float32)
    cnt_ge, cnt_gt, sum_gt = zero, zero, zero
    for g in range(NCHUNK):
        if g + 1 < NCHUNK:
            pending[(g + 1) % 2] = start(g + 1)
        for h in pending[g % 2]:
            h.wait()
        pb, gb, mb, _ = bufs[g % 2]

        def step(i, carry, pb=pb, gb=gb, mb=mb):
            cge, cgt, sgt = carry
            sl = pl.ds(i * L, L)
            l = jnp.abs(pb[sl] - gb[sl])
            v = l - l * mb[sl]
            cge = cge + jnp.where(v >= thr, one, zero)
            is_gt = v > thr
            cgt = cgt + jnp.where(is_gt, one, zero)
            sgt = sgt + jnp.where(is_gt, v, zero)
            return (cge, cgt, sgt)

        cnt_ge, cnt_gt, sum_gt = lax.fori_loop(
            0, CHUNK // L, step, (cnt_ge, cnt_gt, sum_gt))

    stage[pl.ds(0, L)] = cnt_ge
    stage[pl.ds(L, L)] = cnt_gt
    stage[pl.ds(2 * L, L)] = sum_gt
    pltpu.sync_copy(stage, out_h.at[wid])


def _probe(pred_f, gt_f, mask_f, bits):
    thr = lax.bitcast_convert_type(bits, jnp.float32)
    out = _count_pass(pred_f, gt_f, mask_f, jnp.full((L,), thr))
    s = out.reshape(NW, 3, L).sum(axis=(0, 2))
    return s[0], s[1], s[2]


def _topk_sum(pred_f, gt_f, mask_f, kf):
    def cond(st):
        lo, hi = st
        return hi - lo > 1

    def body(st):
        lo, hi = st
        mid = lo + (hi - lo) // 2
        cge, _, _ = _probe(pred_f, gt_f, mask_f, mid)
        big = cge >= kf
        return (jnp.where(big, mid, lo), jnp.where(big, hi, mid))

    lo0 = jnp.int32(0)
    hi0 = jnp.int32(0x7F800000)
    lo, _ = lax.while_loop(cond, body, (lo0, hi0))
    _, cgt, sgt = _probe(pred_f, gt_f, mask_f, lo)
    t = lax.bitcast_convert_type(lo, jnp.float32)
    return sgt + (kf - cgt) * t


def kernel(pred, gt, mask):
    pred_f = pred.reshape(-1)
    gt_f = gt.reshape(-1)
    mask_f = mask.reshape(-1)

    part = _reduce_pass(pred_f, gt_f, mask_f).reshape(NW, 3, L)
    sums = part.sum(axis=(0, 2))
    pos_sum, pos_cnt, loss_sum = sums[0], sums[1], sums[2]
    neg_sum = loss_sum - pos_sum
    neg_cnt = jnp.float32(N) - pos_cnt
    kf = jnp.minimum(neg_cnt, jnp.floor(pos_cnt * 3.0))

    neg_topk = lax.cond(
        kf >= neg_cnt,
        lambda: neg_sum,
        lambda: _topk_sum(pred_f, gt_f, mask_f, kf),
    )

    positive_loss = pos_sum / pos_cnt
    negative_loss = neg_topk / kf
    return (positive_loss + negative_loss, positive_loss, negative_loss)


# speedup vs baseline: 80.2402x; 1.7160x over previous
import functools

import jax
import jax.numpy as jnp
from jax import lax
from jax.experimental import pallas as pl
from jax.experimental.pallas import tpu as pltpu
from jax.experimental.pallas import tpu_sc as plsc

NC = 2
NS = 16
L = 16
NW = NC * NS

N = 16 * 512 * 512
W = 512
ROWS = N // W
R_PER_W = ROWS // NW
CR = 32
NCHUNK = R_PER_W // CR

_MESH = plsc.VectorSubcoreMesh(core_axis_name="c", subcore_axis_name="s")
_PARAMS = pltpu.CompilerParams(use_tc_tiling_on_sc=True)


def _tree_add(vs):
    while len(vs) > 1:
        vs = [a + b for a, b in zip(vs[::2], vs[1::2])] + (
            [vs[-1]] if len(vs) % 2 else [])
    return vs[0]


@functools.partial(
    pl.kernel,
    mesh=_MESH,
    out_type=jax.ShapeDtypeStruct((NW, 3 * L), jnp.float32),
    compiler_params=_PARAMS,
    scratch_types=(
        [pltpu.VMEM((CR, W), jnp.float32) for _ in range(6)]
        + [pltpu.VMEM((3 * L,), jnp.float32)]
        + [pltpu.SemaphoreType.DMA, pltpu.SemaphoreType.DMA]
    ),
)
def _reduce_pass(pred_h, gt_h, mask_h, out_h,
                 p0, g0, m0, p1, g1, m1, stage, sem0, sem1):
    wid = lax.axis_index("c") * NS + lax.axis_index("s")
    base = wid * R_PER_W
    bufs = ((p0, g0, m0, sem0), (p1, g1, m1, sem1))

    def start(g):
        pb, gb, mb, sem = bufs[g % 2]
        off = base + g * CR
        return (
            pltpu.async_copy(pred_h.at[pl.ds(off, CR)], pb, sem),
            pltpu.async_copy(gt_h.at[pl.ds(off, CR)], gb, sem),
            pltpu.async_copy(mask_h.at[pl.ds(off, CR)], mb, sem),
        )

    pending = [None, None]
    pending[0] = start(0)
    zero = jnp.zeros((L,), jnp.float32)
    pos_sum, pos_cnt, loss_sum = zero, zero, zero
    for g in range(NCHUNK):
        if g + 1 < NCHUNK:
            pending[(g + 1) % 2] = start(g + 1)
        for h in pending[g % 2]:
            h.wait()
        pb, gb, mb, _ = bufs[g % 2]

        def step(r, carry, pb=pb, gb=gb, mb=mb):
            ps, pc, ls = carry
            lv, mv, pv = [], [], []
            for c in range(0, W, L):
                sl = pl.ds(c, L)
                l = jnp.abs(pb[r, sl] - gb[r, sl])
                mk = mb[r, sl]
                lv.append(l)
                mv.append(mk)
                pv.append(l * mk)
            return (ps + _tree_add(pv), pc + _tree_add(mv), ls + _tree_add(lv))

        pos_sum, pos_cnt, loss_sum = lax.fori_loop(
            0, CR, step, (pos_sum, pos_cnt, loss_sum))

    stage[pl.ds(0, L)] = pos_sum
    stage[pl.ds(L, L)] = pos_cnt
    stage[pl.ds(2 * L, L)] = loss_sum
    pltpu.sync_copy(stage, out_h.at[wid])


@functools.partial(
    pl.kernel,
    mesh=_MESH,
    out_type=jax.ShapeDtypeStruct((NW, 3 * L), jnp.float32),
    compiler_params=_PARAMS,
    scratch_types=(
        [pltpu.VMEM((CR, W), jnp.float32) for _ in range(6)]
        + [pltpu.VMEM((3 * L,), jnp.float32), pltpu.VMEM((8, 128), jnp.float32)]
        + [pltpu.SemaphoreType.DMA, pltpu.SemaphoreType.DMA]
    ),
)
def _count_pass(pred_h, gt_h, mask_h, thr_h, out_h,
                p0, g0, m0, p1, g1, m1, stage, thr_v, sem0, sem1):
    wid = lax.axis_index("c") * NS + lax.axis_index("s")
    base = wid * R_PER_W
    pltpu.sync_copy(thr_h, thr_v)
    thr = thr_v[0, pl.ds(0, L)]
    bufs = ((p0, g0, m0, sem0), (p1, g1, m1, sem1))

    def start(g):
        pb, gb, mb, sem = bufs[g % 2]
        off = base + g * CR
        return (
            pltpu.async_copy(pred_h.at[pl.ds(off, CR)], pb, sem),
            pltpu.async_copy(gt_h.at[pl.ds(off, CR)], gb, sem),
            pltpu.async_copy(mask_h.at[pl.ds(off, CR)], mb, sem),
        )

    pending = [None, None]
    pending[0] = start(0)
    zero = jnp.zeros((L,), jnp.float32)
    one = jnp.ones((L,), jnp.float32)
    cnt_ge, cnt_gt, sum_gt = zero, zero, zero
    for g in range(NCHUNK):
        if g + 1 < NCHUNK:
            pending[(g + 1) % 2] = start(g + 1)
        for h in pending[g % 2]:
            h.wait()
        pb, gb, mb, _ = bufs[g % 2]

        def step(r, carry, pb=pb, gb=gb, mb=mb):
            cge, cgt, sgt = carry
            for c in range(0, W, L):
                sl = pl.ds(c, L)
                l = jnp.abs(pb[r, sl] - gb[r, sl])
                v = l - l * mb[r, sl]
                cge = cge + jnp.where(v >= thr, one, zero)
                is_gt = v > thr
                cgt = cgt + jnp.where(is_gt, one, zero)
                sgt = sgt + jnp.where(is_gt, v, zero)
            return (cge, cgt, sgt)

        cnt_ge, cnt_gt, sum_gt = lax.fori_loop(
            0, CR, step, (cnt_ge, cnt_gt, sum_gt))

    stage[pl.ds(0, L)] = cnt_ge
    stage[pl.ds(L, L)] = cnt_gt
    stage[pl.ds(2 * L, L)] = sum_gt
    pltpu.sync_copy(stage, out_h.at[wid])


def _probe(pred_t, gt_t, mask_t, bits):
    thr = lax.bitcast_convert_type(bits, jnp.float32)
    out = _count_pass(pred_t, gt_t, mask_t, jnp.full((8, 128), thr))
    s = out.reshape(NW, 3, L).sum(axis=(0, 2))
    return s[0], s[1], s[2]


def _topk_sum(pred_t, gt_t, mask_t, kf):
    def cond(st):
        lo, hi = st
        return hi - lo > 1

    def body(st):
        lo, hi = st
        mid = lo + (hi - lo) // 2
        cge, _, _ = _probe(pred_t, gt_t, mask_t, mid)
        big = cge >= kf
        return (jnp.where(big, mid, lo), jnp.where(big, hi, mid))

    lo0 = jnp.int32(0)
    hi0 = jnp.int32(0x7F800000)
    lo, _ = lax.while_loop(cond, body, (lo0, hi0))
    _, cgt, sgt = _probe(pred_t, gt_t, mask_t, lo)
    t = lax.bitcast_convert_type(lo, jnp.float32)
    return sgt + (kf - cgt) * t


def kernel(pred, gt, mask):
    pred_t = pred.reshape(ROWS, W)
    gt_t = gt.reshape(ROWS, W)
    mask_t = mask.reshape(ROWS, W)

    part = _reduce_pass(pred_t, gt_t, mask_t).reshape(NW, 3, L)
    sums = part.sum(axis=(0, 2))
    pos_sum, pos_cnt, loss_sum = sums[0], sums[1], sums[2]
    neg_sum = loss_sum - pos_sum
    neg_cnt = jnp.float32(N) - pos_cnt
    kf = jnp.minimum(neg_cnt, jnp.floor(pos_cnt * 3.0))

    neg_topk = lax.cond(
        kf >= neg_cnt,
        lambda: neg_sum,
        lambda: _topk_sum(pred_t, gt_t, mask_t, kf),
    )

    positive_loss = pos_sum / pos_cnt
    negative_loss = neg_topk / kf
    return (positive_loss + negative_loss, positive_loss, negative_loss)
